# Initial kernel scaffold; baseline (speedup 1.0000x reference)
import jax
import jax.numpy as jnp
from jax.experimental import pallas as pl


def kernel(x_task, x_dep, edge_index_tt, edge_index_dt, W):
    raise NotImplementedError("write your pallas kernel here")


# speedup vs baseline: 7.8097x; 7.8097x over previous
import functools

import jax
import jax.numpy as jnp
from jax import lax
from jax.experimental import pallas as pl
from jax.experimental.pallas import tpu as pltpu
from jax.experimental.pallas import tpu_sc as plsc

N = 10000
D = 128
E = 320000
NPAD = 10240
NC = 2
NS = 16
NW = NC * NS
EPW = E // NW
CHK = 80
NCH = EPW // CHK
RPT = NPAD // NS

_MESH = dict(core_axis_name="c", subcore_axis_name="s", num_cores=NC,
             num_subcores=NS)


def _hist_body(r_tt, c_tt, r_dt, c_dt, zeros1, hist, acc, idx_v, ones_v):
    cid = lax.axis_index("c")
    sid = lax.axis_index("s")
    ebase = (cid * NS + sid) * EPW
    r0 = sid * RPT
    for i in range(CHK // 16):
        ones_v[pl.ds(i * 16, 16)] = jnp.full((16,), 1.0, jnp.float32)
    for j, idx_hbm in enumerate((r_tt, c_tt, r_dt, c_dt)):
        pltpu.sync_copy(zeros1.at[pl.ds(r0, RPT)], acc.at[pl.ds(r0, RPT)])
        plsc.subcore_barrier()

        def chunk(k, _):
            base = ebase + k * CHK
            pltpu.sync_copy(idx_hbm.at[pl.ds(base, CHK)], idx_v)
            pltpu.sync_copy(ones_v, acc.at[idx_v], add=True)
            return _

        lax.fori_loop(0, NCH, chunk, None)
        plsc.subcore_barrier()
        pltpu.sync_copy(acc.at[pl.ds(r0, RPT)],
                        hist.at[j, cid, pl.ds(r0, RPT)])
        plsc.subcore_barrier()


def _sc_hist(row_tt, col_tt, row_dt, col_dt, zeros1):
    return pl.kernel(
        _hist_body,
        out_type=jax.ShapeDtypeStruct((4, NC, NPAD), jnp.float32),
        mesh=plsc.VectorSubcoreMesh(**_MESH),
        scratch_types=[
            pltpu.VMEM_SHARED((NPAD,), jnp.float32),
            pltpu.VMEM((CHK,), jnp.int32),
            pltpu.VMEM((CHK,), jnp.float32),
        ],
    )(row_tt, col_tt, row_dt, col_dt, zeros1)


def _prop_body(nprop, *refs):
    srcs = refs[:nprop]
    r_tt, c_tt, r_dt, c_dt, zeros2 = refs[nprop:nprop + 5]
    outs = refs[nprop + 5:2 * nprop + 5]
    acc, gidx_v, sidx_v, rows_v, sem = refs[2 * nprop + 5:]
    gs = ((c_tt, r_tt), (r_tt, c_tt), (c_dt, r_dt), (r_dt, c_dt))[:nprop]
    cid = lax.axis_index("c")
    sid = lax.axis_index("s")
    ebase = (cid * NS + sid) * EPW
    r0 = sid * RPT
    for src, (g_hbm, s_hbm), out in zip(srcs, gs, outs):
        pltpu.sync_copy(zeros2.at[pl.ds(r0, RPT)], acc.at[pl.ds(r0, RPT)])
        plsc.subcore_barrier()

        def chunk(k, _):
            base = ebase + k * CHK
            pltpu.sync_copy(g_hbm.at[pl.ds(base, CHK)], gidx_v)
            pltpu.sync_copy(s_hbm.at[pl.ds(base, CHK)], sidx_v)
            pltpu.async_copy(src.at[gidx_v], rows_v, sem).wait()
            pltpu.sync_copy(rows_v, acc.at[sidx_v], add=True)
            return _

        lax.fori_loop(0, NCH, chunk, None)
        plsc.subcore_barrier()
        pltpu.sync_copy(acc.at[pl.ds(r0, RPT)],
                        out.at[cid, pl.ds(r0, RPT)])
        plsc.subcore_barrier()


def _sc_prop(nprop, srcs, idxs, zeros2):
    out_t = tuple(jax.ShapeDtypeStruct((NC, NPAD, D), jnp.float32)
                  for _ in range(nprop))
    return pl.kernel(
        functools.partial(_prop_body, nprop),
        out_type=out_t,
        mesh=plsc.VectorSubcoreMesh(**_MESH),
        scratch_types=[
            pltpu.VMEM_SHARED((NPAD, D), jnp.float32),
            pltpu.VMEM((CHK,), jnp.int32),
            pltpu.VMEM((CHK,), jnp.int32),
            pltpu.VMEM((CHK, D), jnp.float32),
            pltpu.SemaphoreType.DMA,
        ],
    )(*srcs, *idxs, zeros2)


RB = 1000


def _prep_body(histT, xt, xd, s_out, xs0, xs1, xs2, xs3):
    h = histT[...]
    m = (lax.broadcasted_iota(jnp.int32, (8, 4), 0) // 2
         == lax.broadcasted_iota(jnp.int32, (8, 4), 1)).astype(jnp.float32)
    deg = jnp.dot(h, m, preferred_element_type=jnp.float32)
    s = lax.rsqrt(jnp.maximum(deg, 1.0))
    s_out[...] = s
    xtb = xt[...]
    xdb = xd[...]
    xs0[...] = s[:, 1:2] * xtb
    xs1[...] = s[:, 0:1] * xtb
    xs2[...] = s[:, 3:4] * xdb
    xs3[...] = s[:, 2:3] * xdb


def _tc_prep(histT, x_task, x_dep):
    xs_sds = jax.ShapeDtypeStruct((N, D), jnp.float32)
    return pl.pallas_call(
        _prep_body,
        grid=(N // RB,),
        in_specs=[
            pl.BlockSpec((RB, 8), lambda i: (i, 0)),
            pl.BlockSpec((RB, D), lambda i: (i, 0)),
            pl.BlockSpec((RB, D), lambda i: (i, 0)),
        ],
        out_specs=[
            pl.BlockSpec((RB, 4), lambda i: (i, 0)),
            pl.BlockSpec((RB, D), lambda i: (i, 0)),
            pl.BlockSpec((RB, D), lambda i: (i, 0)),
            pl.BlockSpec((RB, D), lambda i: (i, 0)),
            pl.BlockSpec((RB, D), lambda i: (i, 0)),
        ],
        out_shape=[jax.ShapeDtypeStruct((N, 4), jnp.float32),
                   xs_sds, xs_sds, xs_sds, xs_sds],
    )(histT, x_task, x_dep)


def _combine1_body(p0, p1, p2, p3, s_ref, w_ref, xs4, xs5):
    s = s_ref[...]
    h = jnp.zeros((RB, D), jnp.float32)
    for j, p in enumerate((p0, p1, p2, p3)):
        pj = (p[0] + p[1]) * s[:, j:j + 1]
        h += jnp.dot(pj, w_ref[0, j // 2, j % 2],
                     preferred_element_type=jnp.float32)
    h *= 0.5
    xs4[...] = s[:, 1:2] * h
    xs5[...] = s[:, 0:1] * h


def _p_spec():
    return pl.BlockSpec((NC, RB, D), lambda i: (0, i, 0))


def _w_spec():
    return pl.BlockSpec((2, 2, 2, D, D), lambda i: (0, 0, 0, 0, 0))


def _tc_combine1(p0, p1, p2, p3, s, w):
    xs_sds = jax.ShapeDtypeStruct((N, D), jnp.float32)
    return pl.pallas_call(
        _combine1_body,
        grid=(N // RB,),
        in_specs=[_p_spec(), _p_spec(), _p_spec(), _p_spec(),
                  pl.BlockSpec((RB, 4), lambda i: (i, 0)), _w_spec()],
        out_specs=[pl.BlockSpec((RB, D), lambda i: (i, 0)),
                   pl.BlockSpec((RB, D), lambda i: (i, 0))],
        out_shape=[xs_sds, xs_sds],
    )(p0, p1, p2, p3, s, w)


def _combine2_body(p4, p5, p2, p3, s_ref, w_ref, out):
    s = s_ref[...]
    acc = jnp.zeros((RB, D), jnp.float32)
    for (p, sj, wa, wb) in ((p4, 0, 0, 0), (p5, 1, 0, 1),
                            (p2, 2, 1, 0), (p3, 3, 1, 1)):
        pj = (p[0] + p[1]) * s[:, sj:sj + 1]
        acc += jnp.dot(pj, w_ref[1, wa, wb],
                       preferred_element_type=jnp.float32)
    out[...] = 0.5 * acc


def _tc_combine2(p4, p5, p2, p3, s, w):
    return pl.pallas_call(
        _combine2_body,
        grid=(N // RB,),
        in_specs=[_p_spec(), _p_spec(), _p_spec(), _p_spec(),
                  pl.BlockSpec((RB, 4), lambda i: (i, 0)), _w_spec()],
        out_specs=pl.BlockSpec((RB, D), lambda i: (i, 0)),
        out_shape=jax.ShapeDtypeStruct((N, D), jnp.float32),
    )(p4, p5, p2, p3, s, w)


def kernel(x_task, x_dep, edge_index_tt, edge_index_dt, W):
    tt = edge_index_tt.astype(jnp.int32)
    dt = edge_index_dt.astype(jnp.int32)
    row_tt, col_tt = tt[0], tt[1]
    row_dt, col_dt = dt[0], dt[1]
    idxs = (row_tt, col_tt, row_dt, col_dt)
    zeros1 = jnp.zeros((NPAD,), jnp.float32)
    zeros2 = jnp.zeros((NPAD, D), jnp.float32)

    hist = _sc_hist(*idxs, zeros1)
    histT = jnp.transpose(hist.reshape(8, NPAD))
    s, xs0, xs1, xs2, xs3 = _tc_prep(histT, x_task, x_dep)
    p0, p1, p2, p3 = _sc_prop(4, (xs0, xs1, xs2, xs3), idxs, zeros2)
    xs4, xs5 = _tc_combine1(p0, p1, p2, p3, s, W)
    p4, p5 = _sc_prop(2, (xs4, xs5), idxs, zeros2)
    return _tc_combine2(p4, p5, p2, p3, s, W)


# speedup vs baseline: 16.1280x; 2.0651x over previous
import functools

import jax
import jax.numpy as jnp
from jax import lax
from jax.experimental import pallas as pl
from jax.experimental.pallas import tpu as pltpu
from jax.experimental.pallas import tpu_sc as plsc

N = 10000
D = 128
E = 320000
NPAD = 10240
NC = 2
NS = 16
NW = NC * NS
EPW = E // NW
CHK = 80
NCH = EPW // CHK
RPT = NPAD // NS

_MESH = dict(core_axis_name="c", subcore_axis_name="s", num_cores=NC,
             num_subcores=NS)


def _hist_body(r_tt, c_tt, r_dt, c_dt, zeros1, hist, acc, idx_v, ones_v, sem):
    cid = lax.axis_index("c")
    sid = lax.axis_index("s")
    wid = cid * NS + sid
    r0 = sid * RPT
    for i in range(CHK // 16):
        ones_v[pl.ds(i * 16, 16)] = jnp.full((16,), 1.0, jnp.float32)
    for j, idx_hbm in enumerate((r_tt, c_tt, r_dt, c_dt)):
        pltpu.sync_copy(zeros1.at[pl.ds(r0, RPT)], acc.at[pl.ds(r0, RPT)])
        pltpu.sync_copy(idx_hbm.at[wid], idx_v)
        plsc.subcore_barrier()

        def fire(k, _):
            pltpu.async_copy(ones_v, acc.at[idx_v.at[k]], sem, add=True)
            return _

        def drain(k, _):
            pltpu.make_async_copy(ones_v, acc.at[idx_v.at[0]], sem).wait()
            return _

        lax.fori_loop(0, NCH, fire, None)
        lax.fori_loop(0, NCH, drain, None)
        plsc.subcore_barrier()
        pltpu.sync_copy(acc.at[pl.ds(r0, RPT)],
                        hist.at[j, cid, pl.ds(r0, RPT)])
        plsc.subcore_barrier()


def _sc_hist(row_tt, col_tt, row_dt, col_dt, zeros1):
    return pl.kernel(
        _hist_body,
        out_type=jax.ShapeDtypeStruct((4, NC, NPAD), jnp.float32),
        mesh=plsc.VectorSubcoreMesh(**_MESH),
        scratch_types=[
            pltpu.VMEM_SHARED((NPAD,), jnp.float32),
            pltpu.VMEM((NCH, CHK), jnp.int32),
            pltpu.VMEM((CHK,), jnp.float32),
            pltpu.SemaphoreType.DMA,
        ],
    )(row_tt, col_tt, row_dt, col_dt, zeros1)


def _prop_body(nprop, *refs):
    srcs = refs[:nprop]
    rt2, ct2, rd2, cd2 = refs[nprop:nprop + 4]
    rt3, ct3, rd3, cd3 = refs[nprop + 4:nprop + 8]
    zeros2 = refs[nprop + 8]
    outs = refs[nprop + 9:2 * nprop + 9]
    acc, gidx_v, sidx_v, rows0, rows1, gsem0, gsem1 = refs[2 * nprop + 9:]
    gs = ((ct2, rt3), (rt2, ct3), (cd2, rd3), (rd2, cd3))[:nprop]
    cid = lax.axis_index("c")
    sid = lax.axis_index("s")
    wid = cid * NS + sid
    r0 = sid * RPT
    for src, (g_hbm, s_hbm), out in zip(srcs, gs, outs):
        for z in range(4):
            zr = r0 + z * (RPT // 4)
            pltpu.sync_copy(zeros2.at[pl.ds(zr, RPT // 4)],
                            acc.at[pl.ds(zr, RPT // 4)])
        pltpu.sync_copy(g_hbm.at[wid], gidx_v)
        pltpu.sync_copy(s_hbm.at[wid], sidx_v)
        plsc.subcore_barrier()

        def gslice(k):
            return gidx_v.at[pl.ds(k * CHK, CHK)]

        pltpu.async_copy(src.at[gslice(0)], rows0, gsem0)

        def group(g, _):
            k1 = 2 * g + 1
            k2 = 2 * g + 2
            pltpu.make_async_copy(src.at[gslice(0)], rows0, gsem0).wait()
            pltpu.async_copy(src.at[gslice(k1)], rows1, gsem1)
            pltpu.sync_copy(rows0, acc.at[sidx_v.at[2 * g]], add=True)
            pltpu.make_async_copy(src.at[gslice(0)], rows1, gsem1).wait()
            pltpu.async_copy(src.at[gslice(k2)], rows0, gsem0)
            pltpu.sync_copy(rows1, acc.at[sidx_v.at[k1]], add=True)
            return _

        lax.fori_loop(0, (NCH - 1) // 2, group, None)
        pltpu.make_async_copy(src.at[gslice(0)], rows0, gsem0).wait()
        pltpu.sync_copy(rows0, acc.at[sidx_v.at[NCH - 1]], add=True)
        plsc.subcore_barrier()
        for z in range(4):
            zr = r0 + z * (RPT // 4)
            pltpu.sync_copy(acc.at[pl.ds(zr, RPT // 4)],
                            out.at[cid, pl.ds(zr, RPT // 4)])
        plsc.subcore_barrier()


def _sc_prop(nprop, srcs, idxs_flat, idxs, zeros2):
    out_t = tuple(jax.ShapeDtypeStruct((NC, NPAD, D), jnp.float32)
                  for _ in range(nprop))
    return pl.kernel(
        functools.partial(_prop_body, nprop),
        out_type=out_t,
        mesh=plsc.VectorSubcoreMesh(**_MESH),
        scratch_types=[
            pltpu.VMEM_SHARED((NPAD, D), jnp.float32),
            pltpu.VMEM((EPW,), jnp.int32),
            pltpu.VMEM((NCH, CHK), jnp.int32),
            pltpu.VMEM((CHK, D), jnp.float32),
            pltpu.VMEM((CHK, D), jnp.float32),
            pltpu.SemaphoreType.DMA,
            pltpu.SemaphoreType.DMA,
        ],
    )(*srcs, *idxs_flat, *idxs, zeros2)


RB = 1000


def _prep_body(histT, xt, xd, s_out, xs0, xs1, xs2, xs3):
    h = histT[...]
    m = (lax.broadcasted_iota(jnp.int32, (8, 4), 0) // 2
         == lax.broadcasted_iota(jnp.int32, (8, 4), 1)).astype(jnp.float32)
    deg = jnp.dot(h, m, preferred_element_type=jnp.float32)
    s = lax.rsqrt(jnp.maximum(deg, 1.0))
    s_out[...] = s
    xtb = xt[...]
    xdb = xd[...]
    xs0[...] = s[:, 1:2] * xtb
    xs1[...] = s[:, 0:1] * xtb
    xs2[...] = s[:, 3:4] * xdb
    xs3[...] = s[:, 2:3] * xdb


def _tc_prep(histT, x_task, x_dep):
    xs_sds = jax.ShapeDtypeStruct((N, D), jnp.float32)
    return pl.pallas_call(
        _prep_body,
        grid=(N // RB,),
        in_specs=[
            pl.BlockSpec((RB, 8), lambda i: (i, 0)),
            pl.BlockSpec((RB, D), lambda i: (i, 0)),
            pl.BlockSpec((RB, D), lambda i: (i, 0)),
        ],
        out_specs=[
            pl.BlockSpec((RB, 4), lambda i: (i, 0)),
            pl.BlockSpec((RB, D), lambda i: (i, 0)),
            pl.BlockSpec((RB, D), lambda i: (i, 0)),
            pl.BlockSpec((RB, D), lambda i: (i, 0)),
            pl.BlockSpec((RB, D), lambda i: (i, 0)),
        ],
        out_shape=[jax.ShapeDtypeStruct((N, 4), jnp.float32),
                   xs_sds, xs_sds, xs_sds, xs_sds],
    )(histT, x_task, x_dep)


def _combine1_body(p0, p1, p2, p3, s_ref, w_ref, xs4, xs5):
    s = s_ref[...]
    h = jnp.zeros((RB, D), jnp.float32)
    for j, p in enumerate((p0, p1, p2, p3)):
        pj = (p[0] + p[1]) * s[:, j:j + 1]
        h += jnp.dot(pj, w_ref[0, j // 2, j % 2],
                     preferred_element_type=jnp.float32)
    h *= 0.5
    xs4[...] = s[:, 1:2] * h
    xs5[...] = s[:, 0:1] * h


def _p_spec():
    return pl.BlockSpec((NC, RB, D), lambda i: (0, i, 0))


def _w_spec():
    return pl.BlockSpec((2, 2, 2, D, D), lambda i: (0, 0, 0, 0, 0))


def _tc_combine1(p0, p1, p2, p3, s, w):
    xs_sds = jax.ShapeDtypeStruct((N, D), jnp.float32)
    return pl.pallas_call(
        _combine1_body,
        grid=(N // RB,),
        in_specs=[_p_spec(), _p_spec(), _p_spec(), _p_spec(),
                  pl.BlockSpec((RB, 4), lambda i: (i, 0)), _w_spec()],
        out_specs=[pl.BlockSpec((RB, D), lambda i: (i, 0)),
                   pl.BlockSpec((RB, D), lambda i: (i, 0))],
        out_shape=[xs_sds, xs_sds],
    )(p0, p1, p2, p3, s, w)


def _combine2_body(p4, p5, p2, p3, s_ref, w_ref, out):
    s = s_ref[...]
    acc = jnp.zeros((RB, D), jnp.float32)
    for (p, sj, wa, wb) in ((p4, 0, 0, 0), (p5, 1, 0, 1),
                            (p2, 2, 1, 0), (p3, 3, 1, 1)):
        pj = (p[0] + p[1]) * s[:, sj:sj + 1]
        acc += jnp.dot(pj, w_ref[1, wa, wb],
                       preferred_element_type=jnp.float32)
    out[...] = 0.5 * acc


def _tc_combine2(p4, p5, p2, p3, s, w):
    return pl.pallas_call(
        _combine2_body,
        grid=(N // RB,),
        in_specs=[_p_spec(), _p_spec(), _p_spec(), _p_spec(),
                  pl.BlockSpec((RB, 4), lambda i: (i, 0)), _w_spec()],
        out_specs=pl.BlockSpec((RB, D), lambda i: (i, 0)),
        out_shape=jax.ShapeDtypeStruct((N, D), jnp.float32),
    )(p4, p5, p2, p3, s, w)


def kernel(x_task, x_dep, edge_index_tt, edge_index_dt, W):
    tt = edge_index_tt.astype(jnp.int32)
    dt = edge_index_dt.astype(jnp.int32)
    row_tt, col_tt = tt[0], tt[1]
    row_dt, col_dt = dt[0], dt[1]
    idxs_flat = tuple(a.reshape(NW, EPW)
                      for a in (row_tt, col_tt, row_dt, col_dt))
    idxs = tuple(a.reshape(NW, NCH, CHK)
                 for a in (row_tt, col_tt, row_dt, col_dt))
    zeros1 = jnp.zeros((NPAD,), jnp.float32)
    zeros2 = jnp.zeros((NPAD, D), jnp.float32)

    hist = _sc_hist(*idxs, zeros1)
    histT = jnp.transpose(hist.reshape(8, NPAD))
    s, xs0, xs1, xs2, xs3 = _tc_prep(histT, x_task, x_dep)
    p0, p1, p2, p3 = _sc_prop(4, (xs0, xs1, xs2, xs3), idxs_flat, idxs,
                              zeros2)
    xs4, xs5 = _tc_combine1(p0, p1, p2, p3, s, W)
    p4, p5 = _sc_prop(2, (xs4, xs5), idxs_flat, idxs, zeros2)
    return _tc_combine2(p4, p5, p2, p3, s, W)
